# baseline (speedup 1.0000x reference)
import functools
import jax
import jax.numpy as jnp
from jax import lax
from jax.experimental import pallas as pl
from jax.experimental.pallas import tpu as pltpu
from jax.experimental.pallas import tpu_sc as plsc

_ELEM_BATCH = 4096
_ELEM_ATOMS = 50
_ELEM_DOUT = 640

_NC = 2
_NS = 16
_NW = _NC * _NS

_TOTAL = _ELEM_BATCH * _ELEM_ATOMS
_PER_W = _TOTAL // _NW
_CHUNK = 64
_NCHUNK = _PER_W // _CHUNK


def _gate_body(idx_hbm, table_hbm, out_hbm, idx_v, rows0, rows1, sem0, sem1):
    wid = lax.axis_index("s") * _NC + lax.axis_index("c")
    base = wid * _PER_W

    pltpu.sync_copy(idx_hbm.at[wid], idx_v)

    def gather(c, buf, sem):
        return pltpu.make_async_copy(table_hbm.at[idx_v.at[c]], buf, sem)

    gather(0, rows0, sem0).start()

    def step(i, carry):
        c = i * 2
        gather(c + 1, rows1, sem1).start()
        gather(c, rows0, sem0).wait()
        pltpu.sync_copy(rows0, out_hbm.at[pl.ds(base + c * _CHUNK, _CHUNK)])

        @pl.when(c + 2 < _NCHUNK)
        def _():
            gather(c + 2, rows0, sem0).start()

        gather(c + 1, rows1, sem1).wait()
        pltpu.sync_copy(rows1, out_hbm.at[pl.ds(base + (c + 1) * _CHUNK, _CHUNK)])
        return carry

    lax.fori_loop(0, _NCHUNK // 2, step, 0)


@jax.jit
def _gate_lookup(idx3, gate_weight):
    mesh = plsc.VectorSubcoreMesh(core_axis_name="c", subcore_axis_name="s")
    run = pl.kernel(
        _gate_body,
        out_type=jax.ShapeDtypeStruct((_TOTAL, _ELEM_DOUT), jnp.float32),
        mesh=mesh,
        scratch_types=[
            pltpu.VMEM((_NCHUNK, _CHUNK), jnp.int32),
            pltpu.VMEM((_CHUNK, _ELEM_DOUT), jnp.float32),
            pltpu.VMEM((_CHUNK, _ELEM_DOUT), jnp.float32),
            pltpu.SemaphoreType.DMA,
            pltpu.SemaphoreType.DMA,
        ],
    )
    return run(idx3, gate_weight)


def kernel(inputs, gate_weight):
    idx3 = inputs.reshape(_NW, _NCHUNK, _CHUNK)
    out = _gate_lookup(idx3, gate_weight)
    return out.reshape(_ELEM_BATCH, _ELEM_ATOMS, _ELEM_DOUT)


# speedup vs baseline: 2.0652x; 2.0652x over previous
import functools
import jax
import jax.numpy as jnp
from jax import lax
from jax.experimental import pallas as pl
from jax.experimental.pallas import tpu as pltpu
from jax.experimental.pallas import tpu_sc as plsc

_BATCH = 4096
_ATOMS = 50
_DOUT = 640
_NROWS = 10

_NC = 2
_NS = 16
_NW = _NC * _NS

_B_PER_W = _BATCH // _NW
_LANES = 16
_VPR = _DOUT // _LANES


def _gate_body(idx_hbm, table_hbm, out_hbm, idx_v, table_v, buf0, buf1,
               wsem0, wsem1):
    wid = lax.axis_index("s") * _NC + lax.axis_index("c")
    base = wid * _B_PER_W

    pltpu.sync_copy(table_hbm, table_v)
    pltpu.sync_copy(idx_hbm.at[pl.ds(base * _ATOMS, _B_PER_W * _ATOMS)],
                    idx_v.at[pl.ds(0, _B_PER_W * _ATOMS)])

    bufs = (buf0, buf1)
    sems = (wsem0, wsem1)

    def build(c, buf):
        def row(r, carry):
            iv = idx_v[pl.ds(c * _ATOMS + r, _LANES)][0]
            for j in range(_VPR):
                buf[r, pl.ds(j * _LANES, _LANES)] = (
                    table_v[iv, pl.ds(j * _LANES, _LANES)])
            return carry
        lax.fori_loop(0, _ATOMS, row, 0)

    def write(c, buf, sem):
        return pltpu.make_async_copy(buf, out_hbm.at[base + c], sem)

    for b in range(2):
        build(b, bufs[b])
        write(b, bufs[b], sems[b]).start()

    def step(i, carry):
        for b in range(2):
            c = 2 + 2 * i + b
            write(c - 2, bufs[b], sems[b]).wait()
            build(c, bufs[b])
            write(c, bufs[b], sems[b]).start()
        return carry

    lax.fori_loop(0, (_B_PER_W - 2) // 2, step, 0)

    for b in range(2):
        write(0, bufs[b], sems[b]).wait()


@jax.jit
def _gate_lookup(inputs, gate_weight):
    mesh = plsc.VectorSubcoreMesh(core_axis_name="c", subcore_axis_name="s")
    run = pl.kernel(
        _gate_body,
        out_type=jax.ShapeDtypeStruct((_BATCH, _ATOMS, _DOUT), jnp.float32),
        mesh=mesh,
        scratch_types=[
            pltpu.VMEM((_B_PER_W * _ATOMS + _LANES,), jnp.int32),
            pltpu.VMEM((_NROWS, _DOUT), jnp.float32),
            pltpu.VMEM((_ATOMS, _DOUT), jnp.float32),
            pltpu.VMEM((_ATOMS, _DOUT), jnp.float32),
            pltpu.SemaphoreType.DMA,
            pltpu.SemaphoreType.DMA,
        ],
    )
    return run(inputs.reshape(-1), gate_weight)


def kernel(inputs, gate_weight):
    return _gate_lookup(inputs, gate_weight)


# speedup vs baseline: 4.3605x; 2.1114x over previous
import functools
import jax
import jax.numpy as jnp
from jax import lax
from jax.experimental import pallas as pl
from jax.experimental.pallas import tpu as pltpu
from jax.experimental.pallas import tpu_sc as plsc

_BATCH = 4096
_ATOMS = 50
_DOUT = 640
_NROWS = 10

_NC = 2
_NS = 16
_NW = _NC * _NS

_B_PER_W = _BATCH // _NW
_LANES = 16
_VPR = _DOUT // _LANES


def _gate_body(idx_hbm, table_hbm, out_hbm, idx_v, table_v, buf0, buf1,
               wsem0, wsem1):
    wid = lax.axis_index("s") * _NC + lax.axis_index("c")
    base = wid * _B_PER_W

    pltpu.sync_copy(table_hbm, table_v)
    pltpu.sync_copy(idx_hbm.at[pl.ds(base * _ATOMS, _B_PER_W * _ATOMS)],
                    idx_v.at[pl.ds(0, _B_PER_W * _ATOMS)])

    bufs = (buf0, buf1)
    sems = (wsem0, wsem1)

    def build(c, buf):
        G = 8

        def row(r, carry):
            iv = idx_v[pl.ds(c * _ATOMS + r, _LANES)][0]
            vals = [table_v[iv, pl.ds(j * _LANES, _LANES)] for j in range(G)]
            for j in range(_VPR):
                buf[r, pl.ds(j * _LANES, _LANES)] = vals[j % G]
                if j + G < _VPR:
                    vals[j % G] = table_v[iv, pl.ds((j + G) * _LANES, _LANES)]
            return carry
        lax.fori_loop(0, _ATOMS, row, 0)

    def write(c, buf, sem):
        return pltpu.make_async_copy(buf, out_hbm.at[base + c], sem)

    for b in range(2):
        build(b, bufs[b])
        write(b, bufs[b], sems[b]).start()

    def step(i, carry):
        for b in range(2):
            c = 2 + 2 * i + b
            write(c - 2, bufs[b], sems[b]).wait()
            build(c, bufs[b])
            write(c, bufs[b], sems[b]).start()
        return carry

    lax.fori_loop(0, (_B_PER_W - 2) // 2, step, 0)

    for b in range(2):
        write(0, bufs[b], sems[b]).wait()


@jax.jit
def _gate_lookup(inputs, gate_weight):
    mesh = plsc.VectorSubcoreMesh(core_axis_name="c", subcore_axis_name="s")
    run = pl.kernel(
        _gate_body,
        out_type=jax.ShapeDtypeStruct((_BATCH, _ATOMS, _DOUT), jnp.float32),
        mesh=mesh,
        scratch_types=[
            pltpu.VMEM((_B_PER_W * _ATOMS + _LANES,), jnp.int32),
            pltpu.VMEM((_NROWS, _DOUT), jnp.float32),
            pltpu.VMEM((_ATOMS, _DOUT), jnp.float32),
            pltpu.VMEM((_ATOMS, _DOUT), jnp.float32),
            pltpu.SemaphoreType.DMA,
            pltpu.SemaphoreType.DMA,
        ],
    )
    return run(inputs.reshape(-1), gate_weight)


def kernel(inputs, gate_weight):
    return _gate_lookup(inputs, gate_weight)
